# baseline (speedup 1.0000x reference)
import functools

import jax
import jax.numpy as jnp
from jax import lax
from jax.experimental import pallas as pl
from jax.experimental.pallas import tpu as pltpu
from jax.experimental.pallas import tpu_sc as plsc

_CHUNK_ROWS = 64


def _sc_copy_rows(table, row_lo, nrows):
    rows, dim = table.shape
    info = plsc.get_sparse_core_info()
    num_workers = info.num_cores * info.num_subcores
    rows_per_worker = nrows // num_workers
    nchunk = rows_per_worker // _CHUNK_ROWS
    mesh = plsc.VectorSubcoreMesh(core_axis_name="c", subcore_axis_name="s")

    @functools.partial(
        pl.kernel,
        mesh=mesh,
        out_type=jax.ShapeDtypeStruct((nrows, dim), table.dtype),
        scratch_types=[
            pltpu.VMEM((_CHUNK_ROWS, dim), jnp.float32),
            pltpu.VMEM((_CHUNK_ROWS, dim), jnp.float32),
            pltpu.SemaphoreType.DMA,
            pltpu.SemaphoreType.DMA,
            pltpu.SemaphoreType.DMA,
            pltpu.SemaphoreType.DMA,
        ],
    )
    def sc_copy(table_hbm, out_hbm, buf0, buf1, li0, li1, so0, so1):
        wid = lax.axis_index("s") * info.num_cores + lax.axis_index("c")
        base = wid * rows_per_worker
        bufs = (buf0, buf1)
        load_sems = (li0, li1)
        store_sems = (so0, so1)

        def start_load(c, b):
            return pltpu.async_copy(
                table_hbm.at[pl.ds(row_lo + base + c * _CHUNK_ROWS, _CHUNK_ROWS)],
                bufs[b],
                load_sems[b],
            )

        def start_store(c, b):
            return pltpu.async_copy(
                bufs[b],
                out_hbm.at[pl.ds(base + c * _CHUNK_ROWS, _CHUNK_ROWS)],
                store_sems[b],
            )

        loads = [None, None]
        stores = [None, None]
        loads[0] = start_load(0, 0)
        for c in range(nchunk):
            b = c & 1
            nb = (c + 1) & 1
            if c + 1 < nchunk:
                if stores[nb] is not None:
                    stores[nb].wait()
                loads[nb] = start_load(c + 1, nb)
            loads[b].wait()
            stores[b] = start_store(c, b)
        for b in range(2):
            if stores[b] is not None:
                stores[b].wait()

    return sc_copy(table)


def _tc_copy_block(in_ref, out_ref):
    out_ref[...] = in_ref[...]


def _tc_copy_rows(table, row_lo, nrows):
    rows, dim = table.shape
    blk = 512
    off = row_lo // blk
    return pl.pallas_call(
        _tc_copy_block,
        grid=(nrows // blk,),
        in_specs=[pl.BlockSpec((blk, dim), lambda i: (i + off, 0))],
        out_specs=pl.BlockSpec((blk, dim), lambda i: (i, 0)),
        out_shape=jax.ShapeDtypeStruct((nrows, dim), table.dtype),
    )(table)


def kernel(seq_len, table):
    del seq_len
    rows, dim = table.shape
    half = rows // 2
    sc_part = _sc_copy_rows(table, 0, half)
    tc_part = _tc_copy_rows(table, half, rows - half)
    return jnp.concatenate([sc_part, tc_part], axis=0)[None]


# speedup vs baseline: 1.2354x; 1.2354x over previous
import functools

import jax
import jax.numpy as jnp
from jax import lax
from jax.experimental import pallas as pl
from jax.experimental.pallas import tpu as pltpu
from jax.experimental.pallas import tpu_sc as plsc

_CHUNK_ROWS = 64


def kernel(seq_len, table):
    del seq_len
    rows, dim = table.shape
    info = plsc.get_sparse_core_info()
    num_workers = info.num_subcores
    rows_per_worker = rows // num_workers
    nchunk = rows_per_worker // _CHUNK_ROWS

    mesh = plsc.VectorSubcoreMesh(
        core_axis_name="c", subcore_axis_name="s", num_cores=1
    )

    @functools.partial(
        pl.kernel,
        mesh=mesh,
        out_type=jax.ShapeDtypeStruct((rows, dim), table.dtype),
        scratch_types=[
            pltpu.VMEM((_CHUNK_ROWS, dim), jnp.float32),
            pltpu.VMEM((_CHUNK_ROWS, dim), jnp.float32),
            pltpu.SemaphoreType.DMA,
            pltpu.SemaphoreType.DMA,
            pltpu.SemaphoreType.DMA,
            pltpu.SemaphoreType.DMA,
        ],
    )
    def sc_copy(table_hbm, out_hbm, buf0, buf1, li0, li1, so0, so1):
        wid = lax.axis_index("s")
        base = wid * rows_per_worker
        bufs = (buf0, buf1)
        load_sems = (li0, li1)
        store_sems = (so0, so1)

        def start_load(c, b):
            return pltpu.async_copy(
                table_hbm.at[pl.ds(base + c * _CHUNK_ROWS, _CHUNK_ROWS)],
                bufs[b],
                load_sems[b],
            )

        def start_store(c, b):
            return pltpu.async_copy(
                bufs[b],
                out_hbm.at[pl.ds(base + c * _CHUNK_ROWS, _CHUNK_ROWS)],
                store_sems[b],
            )

        loads = [None, None]
        stores = [None, None]
        loads[0] = start_load(0, 0)
        for c in range(nchunk):
            b = c & 1
            nb = (c + 1) & 1
            if c + 1 < nchunk:
                if stores[nb] is not None:
                    stores[nb].wait()
                loads[nb] = start_load(c + 1, nb)
            loads[b].wait()
            stores[b] = start_store(c, b)
        for b in range(2):
            if stores[b] is not None:
                stores[b].wait()

    out = sc_copy(table)
    return out[None]


# speedup vs baseline: 1.3857x; 1.1216x over previous
import functools

import jax
import jax.numpy as jnp
from jax import lax
from jax.experimental import pallas as pl
from jax.experimental.pallas import tpu as pltpu
from jax.experimental.pallas import tpu_sc as plsc

_CHUNK_ROWS = 64


def kernel(seq_len, table):
    del seq_len
    rows, dim = table.shape
    info = plsc.get_sparse_core_info()
    num_workers = info.num_cores * info.num_subcores
    rows_per_worker = rows // num_workers
    nchunk = rows_per_worker // _CHUNK_ROWS

    mesh = plsc.VectorSubcoreMesh(core_axis_name="c", subcore_axis_name="s")

    @functools.partial(
        pl.kernel,
        mesh=mesh,
        out_type=jax.ShapeDtypeStruct((rows, dim), table.dtype),
        scratch_types=[
            pltpu.VMEM((_CHUNK_ROWS, dim), jnp.float32),
            pltpu.VMEM((_CHUNK_ROWS, dim), jnp.float32),
            pltpu.SemaphoreType.DMA,
            pltpu.SemaphoreType.DMA,
            pltpu.SemaphoreType.DMA,
            pltpu.SemaphoreType.DMA,
        ],
    )
    def sc_copy(table_hbm, out_hbm, buf0, buf1, li0, li1, so0, so1):
        wid = lax.axis_index("s") * info.num_cores + lax.axis_index("c")
        base = wid * rows_per_worker
        bufs = (buf0, buf1)
        load_sems = (li0, li1)
        store_sems = (so0, so1)

        def start_load(c, b):
            return pltpu.async_copy(
                table_hbm.at[pl.ds(base + c * _CHUNK_ROWS, _CHUNK_ROWS)],
                bufs[b],
                load_sems[b],
            )

        def start_store(c, b):
            return pltpu.async_copy(
                bufs[b],
                out_hbm.at[pl.ds(base + c * _CHUNK_ROWS, _CHUNK_ROWS)],
                store_sems[b],
            )

        loads = [None, None]
        stores = [None, None]
        loads[0] = start_load(0, 0)
        for c in range(nchunk):
            b = c & 1
            nb = (c + 1) & 1
            if c + 1 < nchunk:
                if stores[nb] is not None:
                    stores[nb].wait()
                loads[nb] = start_load(c + 1, nb)
            loads[b].wait()
            stores[b] = start_store(c, b)
        for b in range(2):
            if stores[b] is not None:
                stores[b].wait()

    out = sc_copy(table)
    return out[None]
